# Initial kernel scaffold; baseline (speedup 1.0000x reference)
import jax
import jax.numpy as jnp
from jax.experimental import pallas as pl


def kernel(edge_index, edge_type, edge_norm, subj, rel, obj, init_embed, init_rel, w_in1, w_out1, w_loop1, w_rel1, loop_rel1, bias1, w_in2, w_out2, w_loop2, w_rel2, loop_rel2, bias2):
    raise NotImplementedError("write your pallas kernel here")


# speedup vs baseline: 3.0479x; 3.0479x over previous
import functools

import jax
import jax.numpy as jnp
from jax import lax
from jax.experimental import pallas as pl
from jax.experimental.pallas import tpu as pltpu
from jax.experimental.pallas import tpu_sc as plsc

NUM_ENT = 10000
N_EDGES = 320000
DIM = 128
NUM_REL2 = 474
R_PAD = 480
NC, NS, L = 2, 16, 16
EDGES_PER_CORE = N_EDGES // NC
EDGES_PER_TILE = EDGES_PER_CORE // NS
CHUNK = 80
N_CHUNKS = EDGES_PER_TILE // CHUNK
STRIPE = 640
N_BOUNCE = STRIPE // CHUNK
BATCH = 4096
GB = BATCH // (NC * NS)


def _edge_pass_body(x_hbm, r_hbm, src_hbm, dst_hbm, et_hbm, nrm_hbm, out_hbm,
                    accum, xbuf, rbuf, sidx, eidx, didx, nrmv,
                    sem1, sem2):
    c = lax.axis_index("c")
    s = lax.axis_index("s")
    base_e = c * EDGES_PER_CORE + s * EDGES_PER_TILE

    zero16 = jnp.zeros((L,), jnp.float32)

    def zrow(j, carry):
        for k in range(DIM // L):
            xbuf[j, pl.ds(L * k, L)] = zero16
        return carry

    lax.fori_loop(0, CHUNK, zrow, 0)
    for t in range(N_BOUNCE):
        ro = s * STRIPE + t * CHUNK

        @pl.when(ro < NUM_ENT)
        def _():
            pltpu.sync_copy(xbuf, accum.at[pl.ds(ro, CHUNK)])

    plsc.subcore_barrier()

    def chunk_body(i, carry):
        eb = base_e + i * CHUNK
        pltpu.sync_copy(src_hbm.at[pl.ds(eb, CHUNK)], sidx)
        pltpu.sync_copy(et_hbm.at[pl.ds(eb, CHUNK)], eidx)
        pltpu.sync_copy(dst_hbm.at[pl.ds(eb, CHUNK)], didx)
        pltpu.sync_copy(nrm_hbm.at[pl.ds(eb, CHUNK)], nrmv)
        cp1 = pltpu.async_copy(x_hbm.at[sidx], xbuf, sem1)
        cp2 = pltpu.async_copy(r_hbm.at[eidx], rbuf, sem2)
        cp1.wait()
        cp2.wait()

        def grp_body(g, gcarry):
            nvec = nrmv[pl.ds(L * g, L)]
            for j in range(L):
                nb = lax.gather(
                    nvec, jnp.full((L, 1), j, jnp.int32),
                    lax.GatherDimensionNumbers(
                        offset_dims=(), collapsed_slice_dims=(0,),
                        start_index_map=(0,)),
                    slice_sizes=(1,),
                    mode=lax.GatherScatterMode.PROMISE_IN_BOUNDS)
                e = g * L + j
                for k in range(DIM // L):
                    sl = pl.ds(L * k, L)
                    xbuf[e, sl] = xbuf[e, sl] * rbuf[e, sl] * nb
            return gcarry

        lax.fori_loop(0, CHUNK // L, grp_body, 0)
        pltpu.sync_copy(xbuf, accum.at[didx], add=True)
        return carry

    lax.fori_loop(0, N_CHUNKS, chunk_body, 0)
    plsc.subcore_barrier()

    for t in range(N_BOUNCE):
        ro = s * STRIPE + t * CHUNK

        @pl.when(ro < NUM_ENT)
        def _():
            pltpu.sync_copy(accum.at[pl.ds(ro, CHUNK)], xbuf)
            pltpu.sync_copy(xbuf, out_hbm.at[pl.ds(c * NUM_ENT + ro, CHUNK)])


_edge_pass = functools.partial(
    pl.kernel,
    out_type=jax.ShapeDtypeStruct((NC * NUM_ENT, DIM), jnp.float32),
    mesh=plsc.VectorSubcoreMesh(core_axis_name="c", subcore_axis_name="s"),
    scratch_types=[
        pltpu.VMEM_SHARED((NUM_ENT, DIM), jnp.float32),
        pltpu.VMEM((CHUNK, DIM), jnp.float32),
        pltpu.VMEM((CHUNK, DIM), jnp.float32),
        pltpu.VMEM((CHUNK,), jnp.int32),
        pltpu.VMEM((CHUNK,), jnp.int32),
        pltpu.VMEM((CHUNK,), jnp.int32),
        pltpu.VMEM((CHUNK,), jnp.float32),
        pltpu.SemaphoreType.DMA,
        pltpu.SemaphoreType.DMA,
    ],
)(_edge_pass_body)


def _dense_body(a0, a1, xb, rb, w_in, w_out, w_loop, w_rel, lr, bias,
                xout, rout):
    acc = jnp.dot(a0[...], w_in[...], preferred_element_type=jnp.float32)
    acc = acc + jnp.dot(a1[...], w_out[...], preferred_element_type=jnp.float32)
    acc = acc + jnp.dot(xb[...] * lr[...], w_loop[...],
                        preferred_element_type=jnp.float32)
    xout[...] = jnp.tanh(acc * (1.0 / 3.0) + bias[...])

    @pl.when(pl.program_id(0) == 0)
    def _():
        rout[...] = jnp.dot(rb[...], w_rel[...],
                            preferred_element_type=jnp.float32)


_BLK = 1000
_dense = pl.pallas_call(
    _dense_body,
    grid=(NUM_ENT // _BLK,),
    in_specs=[
        pl.BlockSpec((_BLK, DIM), lambda i: (i, 0)),
        pl.BlockSpec((_BLK, DIM), lambda i: (i, 0)),
        pl.BlockSpec((_BLK, DIM), lambda i: (i, 0)),
        pl.BlockSpec((R_PAD, DIM), lambda i: (0, 0)),
        pl.BlockSpec((DIM, DIM), lambda i: (0, 0)),
        pl.BlockSpec((DIM, DIM), lambda i: (0, 0)),
        pl.BlockSpec((DIM, DIM), lambda i: (0, 0)),
        pl.BlockSpec((DIM, DIM), lambda i: (0, 0)),
        pl.BlockSpec((1, DIM), lambda i: (0, 0)),
        pl.BlockSpec((1, DIM), lambda i: (0, 0)),
    ],
    out_specs=[
        pl.BlockSpec((_BLK, DIM), lambda i: (i, 0)),
        pl.BlockSpec((R_PAD, DIM), lambda i: (0, 0)),
    ],
    out_shape=[
        jax.ShapeDtypeStruct((NUM_ENT, DIM), jnp.float32),
        jax.ShapeDtypeStruct((R_PAD, DIM), jnp.float32),
    ],
)


def _gather_body(x_hbm, r_hbm, subj_hbm, rel_hbm, obj_hbm,
                 sub_out, rel_out, obj_out, idxv, buf, sem):
    c = lax.axis_index("c")
    s = lax.axis_index("s")
    b = (s * NC + c) * GB
    for tab, idx_h, out_h in ((x_hbm, subj_hbm, sub_out),
                              (r_hbm, rel_hbm, rel_out),
                              (x_hbm, obj_hbm, obj_out)):
        pltpu.sync_copy(idx_h.at[pl.ds(b, GB)], idxv)
        pltpu.async_copy(tab.at[idxv], buf, sem).wait()
        pltpu.sync_copy(buf, out_h.at[pl.ds(b, GB)])


_gather = functools.partial(
    pl.kernel,
    out_type=[
        jax.ShapeDtypeStruct((BATCH, DIM), jnp.float32),
        jax.ShapeDtypeStruct((BATCH, DIM), jnp.float32),
        jax.ShapeDtypeStruct((BATCH, DIM), jnp.float32),
    ],
    mesh=plsc.VectorSubcoreMesh(core_axis_name="c", subcore_axis_name="s"),
    scratch_types=[
        pltpu.VMEM((GB,), jnp.int32),
        pltpu.VMEM((GB, DIM), jnp.float32),
        pltpu.SemaphoreType.DMA,
    ],
)(_gather_body)


def kernel(edge_index, edge_type, edge_norm, subj, rel, obj, init_embed,
           init_rel, w_in1, w_out1, w_loop1, w_rel1, loop_rel1, bias1,
           w_in2, w_out2, w_loop2, w_rel2, loop_rel2, bias2):
    src = edge_index[0].astype(jnp.int32)
    dst = edge_index[1].astype(jnp.int32)
    et = edge_type.astype(jnp.int32)
    subj = subj.astype(jnp.int32)
    rel = rel.astype(jnp.int32)
    obj = obj.astype(jnp.int32)
    nrm = edge_norm.astype(jnp.float32)

    r0 = jnp.concatenate(
        [init_rel, jnp.zeros((R_PAD - NUM_REL2, DIM), jnp.float32)], axis=0)
    b1 = bias1.reshape(1, DIM)
    b2 = bias2.reshape(1, DIM)

    a1 = _edge_pass(init_embed, r0, src, dst, et, nrm)
    x1, r1 = _dense(a1[:NUM_ENT], a1[NUM_ENT:], init_embed, r0,
                    w_in1, w_out1, w_loop1, w_rel1, loop_rel1, b1)
    a2 = _edge_pass(x1, r1, src, dst, et, nrm)
    x2, r2 = _dense(a2[:NUM_ENT], a2[NUM_ENT:], x1, r1,
                    w_in2, w_out2, w_loop2, w_rel2, loop_rel2, b2)
    sub_emb, rel_emb, obj_emb = _gather(x2, r2, subj, rel, obj)
    return sub_emb, rel_emb, obj_emb, x2, r2[:NUM_REL2]
